# Initial kernel scaffold; baseline (speedup 1.0000x reference)
import jax
import jax.numpy as jnp
from jax.experimental import pallas as pl


def kernel(x, edge1, pos, idx, ei2, emb, gn0_w, gn0_b, gn0_ms, W1, b1, gn1_w, gn1_b, gn1_ms, W2, b2, gn2_w, gn2_b, gn2_ms, W2r, b2r, gn2r_w, gn2r_b, gn2r_ms, Wp, bp):
    raise NotImplementedError("write your pallas kernel here")


# speedup vs baseline: 24.6586x; 24.6586x over previous
import functools

import jax
import jax.numpy as jnp
from jax import lax
from jax.experimental import pallas as pl
from jax.experimental.pallas import tpu as pltpu
from jax.experimental.pallas import tpu_sc as plsc

N = 50000
E = 800000
P = 100000
K = 65536
C1 = 64
C2 = 32

NC = 2
NS = 16
NW = NC * NS

NRp = 51200
PRp = 100352
EP = 819200
BN = 53248
BP = 102400
EPS = 1e-5

_mesh = plsc.VectorSubcoreMesh(
    core_axis_name="c", subcore_axis_name="s", num_cores=NC, num_subcores=NS)
_sc_params = pltpu.CompilerParams(use_tc_tiling_on_sc=False)


def _make_gather(T, D, B, G, GG):
  bp = B // NW
  assert bp == G * 128 and G % GG == 0

  @functools.partial(
      pl.kernel,
      out_type=jax.ShapeDtypeStruct((B, D), jnp.float32),
      mesh=_mesh,
      compiler_params=_sc_params,
      scratch_types=[
          pltpu.VMEM((G, 128), jnp.int32),
          pltpu.VMEM((bp, D), jnp.float32),
          pltpu.SemaphoreType.DMA,
      ],
  )
  def k(table, idx3d, out, idxv, rows, sem):
    c = lax.axis_index("c")
    s = lax.axis_index("s")
    wid = c * NS + s
    pltpu.sync_copy(idx3d.at[wid], idxv)

    def grp(ii, _):
      for j in range(GG):
        jj = ii * GG + j
        pltpu.async_copy(table.at[idxv.at[jj]],
                         rows.at[pl.ds(jj * 128, 128)], sem)
      for j in range(GG):
        jj = ii * GG + j
        pltpu.make_async_copy(table.at[idxv.at[jj]],
                              rows.at[pl.ds(jj * 128, 128)], sem).wait()
      return 0

    lax.fori_loop(0, G // GG, grp, 0)
    pltpu.sync_copy(rows, out.at[pl.ds(wid * bp, bp)])

  return k


@functools.partial(
    pl.kernel,
    out_type=(
        jax.ShapeDtypeStruct((NC, NRp), jnp.float32),
        jax.ShapeDtypeStruct((NC, PRp), jnp.float32),
        jax.ShapeDtypeStruct((NC, PRp), jnp.float32),
    ),
    mesh=_mesh,
    compiler_params=_sc_params,
    scratch_types=[
        pltpu.VMEM_SHARED((NRp,), jnp.float32),
        pltpu.VMEM_SHARED((PRp,), jnp.float32),
        pltpu.VMEM_SHARED((PRp,), jnp.float32),
        pltpu.VMEM((8, 128), jnp.int32),
        pltpu.VMEM((128,), jnp.float32),
        pltpu.SemaphoreType.DMA,
    ],
)
def _deg_kernel(d1, d2, d3, zeros1d, p1, p2, p3, a1, a2, a3,
                didx, ones, sem):
  c = lax.axis_index("c")
  s = lax.axis_index("s")
  wid = c * NS + s
  n1 = NRp // NS
  n2 = PRp // NS
  for i in range(8):
    ones[pl.ds(i * 16, 16)] = jnp.ones((16,), jnp.float32)
  pltpu.sync_copy(zeros1d.at[pl.ds(s * n1, n1)], a1.at[pl.ds(s * n1, n1)])
  pltpu.sync_copy(zeros1d.at[pl.ds(s * n2, n2)], a2.at[pl.ds(s * n2, n2)])
  pltpu.sync_copy(zeros1d.at[pl.ds(s * n2, n2)], a3.at[pl.ds(s * n2, n2)])
  plsc.subcore_barrier()

  nrow_pt = EP // 128 // NW

  for (dref, aref) in ((d1, a1), (d2, a2), (d3, a3)):
    def chunk(g, _, dref=dref, aref=aref):
      rb = wid * nrow_pt + g * 8
      pltpu.sync_copy(dref.at[pl.ds(rb, 8)], didx)
      for j in range(8):
        pltpu.async_copy(ones, aref.at[didx.at[j]], sem, add=True)
      for j in range(8):
        pltpu.make_async_copy(ones, aref.at[didx.at[j]], sem).wait()
      return 0

    lax.fori_loop(0, nrow_pt // 8, chunk, 0)

  plsc.subcore_barrier()
  pltpu.sync_copy(a1.at[pl.ds(s * n1, n1)], p1.at[c, pl.ds(s * n1, n1)])
  pltpu.sync_copy(a2.at[pl.ds(s * n2, n2)], p2.at[c, pl.ds(s * n2, n2)])
  pltpu.sync_copy(a3.at[pl.ds(s * n2, n2)], p3.at[c, pl.ds(s * n2, n2)])


def _make_msg(nrows):
  rpt = nrows // NS
  nrow_pt = EP // 128 // NS

  @functools.partial(
      pl.kernel,
      out_type=jax.ShapeDtypeStruct((NC, nrows, 16), jnp.float32),
      mesh=_mesh,
      compiler_params=_sc_params,
      scratch_types=[
          pltpu.VMEM_SHARED((nrows, 16), jnp.float32),
          pltpu.VMEM((8, 128), jnp.int32),
          pltpu.VMEM((8, 128), jnp.int32),
          pltpu.VMEM((1024, 16), jnp.float32),
          pltpu.SemaphoreType.DMA,
          pltpu.SemaphoreType.DMA,
      ],
  )
  def k(src2d, dst2d, table0, table1, zeros, out, acc,
        sidx, didx, msg, semg, sems):
    c = lax.axis_index("c")
    s = lax.axis_index("s")
    pltpu.sync_copy(zeros.at[pl.ds(s * rpt, rpt)], acc.at[pl.ds(s * rpt, rpt)])
    plsc.subcore_barrier()

    def edge_loop(table):
      def chunk(g, _):
        rb = s * nrow_pt + g * 8
        pltpu.sync_copy(src2d.at[pl.ds(rb, 8)], sidx)
        pltpu.sync_copy(dst2d.at[pl.ds(rb, 8)], didx)
        for j in range(8):
          pltpu.async_copy(table.at[sidx.at[j]],
                           msg.at[pl.ds(j * 128, 128)], semg)
        for j in range(8):
          pltpu.make_async_copy(table.at[sidx.at[j]],
                                msg.at[pl.ds(j * 128, 128)], semg).wait()
        for j in range(8):
          pltpu.async_copy(msg.at[pl.ds(j * 128, 128)],
                           acc.at[didx.at[j]], sems, add=True)
        for j in range(8):
          pltpu.make_async_copy(msg.at[pl.ds(j * 128, 128)],
                                acc.at[didx.at[j]], sems).wait()
        return 0

      lax.fori_loop(0, nrow_pt // 8, chunk, 0)

    @pl.when(c == 0)
    def _():
      edge_loop(table0)

    @pl.when(c == 1)
    def _():
      edge_loop(table1)

    plsc.subcore_barrier()
    pltpu.sync_copy(acc.at[pl.ds(s * rpt, rpt)],
                    out.at[c, pl.ds(s * rpt, rpt)])

  return k


def _dis_body(d1, d2, d3, o1, o2, o3):
  o1[...] = lax.rsqrt(d1[0] + d1[1] + 1.0)
  o2[...] = lax.rsqrt(d2[0] + d2[1] + 1.0)
  o3[...] = lax.rsqrt(d3[0] + d3[1] + 1.0)


def _dis_call(p1, p2, p3):
  r1 = NRp // 128
  r2 = PRp // 128
  return pl.pallas_call(
      _dis_body,
      out_shape=(
          jax.ShapeDtypeStruct((r1, 128), jnp.float32),
          jax.ShapeDtypeStruct((r2, 128), jnp.float32),
          jax.ShapeDtypeStruct((r2, 128), jnp.float32),
      ),
  )(p1.reshape(NC, r1, 128), p2.reshape(NC, r2, 128), p3.reshape(NC, r2, 128))


def _stats_body(count, xr, st):
  i = pl.program_id(0)
  blk = xr[...]
  rows = i * 1024 + lax.broadcasted_iota(jnp.int32, (1024, 1), 0)
  m = (rows < count).astype(jnp.float32)
  xm = blk * m
  s1 = jnp.sum(xm, axis=0, keepdims=True)
  s2 = jnp.sum(xm * xm, axis=0, keepdims=True)

  @pl.when(i == 0)
  def _():
    st[...] = jnp.zeros_like(st)

  st[0:1, :] += s1
  st[1:2, :] += s2


def _stats_call(xr, count, C):
  nb = xr.shape[0] // 1024
  return pl.pallas_call(
      functools.partial(_stats_body, count),
      grid=(nb,),
      in_specs=[pl.BlockSpec((1024, C), lambda i: (i, 0))],
      out_specs=pl.BlockSpec((8, C), lambda i: (0, 0)),
      out_shape=jax.ShapeDtypeStruct((8, C), jnp.float32),
  )(xr)


def _gn_apply(x, st, w, b, ms, count):
  m = st[0:1, :] / count
  msq = st[1:2, :] / count
  var = msq - 2.0 * ms * m * m + (ms * m) * (ms * m)
  return w * (x - ms * m) * lax.rsqrt(var + EPS) + b


def _pack1_body(h0, st0, dis, W1, g0w, g0b, g0ms, t1):
  hn = _gn_apply(h0[...], st0[...], g0w[...], g0b[...], g0ms[...], N)
  hW = jnp.dot(hn, W1[...], preferred_element_type=jnp.float32)
  hs = dis[...] * hW
  t1[0] = hs[:, :16]
  t1[1] = hs[:, 16:]


def _pack1_call(h0, st0, dis1, W1, g0w, g0b, g0ms):
  nb = NRp // 1024
  return pl.pallas_call(
      _pack1_body,
      grid=(nb,),
      in_specs=[
          pl.BlockSpec((1024, C1), lambda i: (i, 0)),
          pl.BlockSpec((8, C1), lambda i: (0, 0)),
          pl.BlockSpec((1024, 1), lambda i: (i, 0)),
          pl.BlockSpec((C1, C2), lambda i: (0, 0)),
          pl.BlockSpec((1, C1), lambda i: (0, 0)),
          pl.BlockSpec((1, C1), lambda i: (0, 0)),
          pl.BlockSpec((1, C1), lambda i: (0, 0)),
      ],
      out_specs=pl.BlockSpec((2, 1024, 16), lambda i: (0, i, 0)),
      out_shape=jax.ShapeDtypeStruct((2, NRp, 16), jnp.float32),
  )(h0, st0, dis1, W1, g0w, g0b, g0ms)


def _postagg_body(count, bias_c, acc, tbl, dis, bias, g, st):
  i = pl.program_id(0)
  acc32 = jnp.concatenate([acc[0], acc[1]], axis=1)
  hs32 = jnp.concatenate([tbl[0], tbl[1]], axis=1)
  gv = dis[...] * (acc32 + hs32) + bias[...]
  g[...] = gv
  rows = i * 1024 + lax.broadcasted_iota(jnp.int32, (1024, 1), 0)
  m = (rows < count).astype(jnp.float32)
  gm = gv * m

  @pl.when(i == 0)
  def _():
    st[...] = jnp.zeros_like(st)

  st[0:1, :] += jnp.sum(gm, axis=0, keepdims=True)
  st[1:2, :] += jnp.sum(gm * gm, axis=0, keepdims=True)


def _postagg_call(acc, tbl, dis, bias, nrows, count):
  nb = nrows // 1024
  return pl.pallas_call(
      functools.partial(_postagg_body, count, None),
      grid=(nb,),
      in_specs=[
          pl.BlockSpec((2, 1024, 16), lambda i: (0, i, 0)),
          pl.BlockSpec((2, 1024, 16), lambda i: (0, i, 0)),
          pl.BlockSpec((1024, 1), lambda i: (i, 0)),
          pl.BlockSpec((1, C2), lambda i: (0, 0)),
      ],
      out_specs=(
          pl.BlockSpec((1024, C2), lambda i: (i, 0)),
          pl.BlockSpec((8, C2), lambda i: (0, 0)),
      ),
      out_shape=(
          jax.ShapeDtypeStruct((nrows, C2), jnp.float32),
          jax.ShapeDtypeStruct((8, C2), jnp.float32),
      ),
  )(acc, tbl, dis, bias)


def _apply_body(count, g, st, w, b, ms, h):
  h[...] = jax.nn.relu(_gn_apply(g[...], st[...], w[...], b[...], ms[...],
                                 count))


def _apply_call(g, st, w, b, ms, nrows, count):
  nb = nrows // 1024
  return pl.pallas_call(
      functools.partial(_apply_body, count),
      grid=(nb,),
      in_specs=[
          pl.BlockSpec((1024, C2), lambda i: (i, 0)),
          pl.BlockSpec((8, C2), lambda i: (0, 0)),
          pl.BlockSpec((1, C2), lambda i: (0, 0)),
          pl.BlockSpec((1, C2), lambda i: (0, 0)),
          pl.BlockSpec((1, C2), lambda i: (0, 0)),
      ],
      out_specs=pl.BlockSpec((1024, C2), lambda i: (i, 0)),
      out_shape=jax.ShapeDtypeStruct((nrows, C2), jnp.float32),
  )(g, st, w, b, ms)


def _pack2_body(za, zb, disf, disr, W2, W2r, tf, tr):
  z = za[...] * zb[...]
  zf = disf[...] * jnp.dot(z, W2[...], preferred_element_type=jnp.float32)
  zr = disr[...] * jnp.dot(z, W2r[...], preferred_element_type=jnp.float32)
  tf[0] = zf[:, :16]
  tf[1] = zf[:, 16:]
  tr[0] = zr[:, :16]
  tr[1] = zr[:, 16:]


def _pack2_call(za, zb, disf, disr, W2, W2r):
  nb = PRp // 1024
  return pl.pallas_call(
      _pack2_body,
      grid=(nb,),
      in_specs=[
          pl.BlockSpec((1024, C2), lambda i: (i, 0)),
          pl.BlockSpec((1024, C2), lambda i: (i, 0)),
          pl.BlockSpec((1024, 1), lambda i: (i, 0)),
          pl.BlockSpec((1024, 1), lambda i: (i, 0)),
          pl.BlockSpec((C2, C2), lambda i: (0, 0)),
          pl.BlockSpec((C2, C2), lambda i: (0, 0)),
      ],
      out_specs=(
          pl.BlockSpec((2, 1024, 16), lambda i: (0, i, 0)),
          pl.BlockSpec((2, 1024, 16), lambda i: (0, i, 0)),
      ),
      out_shape=(
          jax.ShapeDtypeStruct((2, PRp, 16), jnp.float32),
          jax.ShapeDtypeStruct((2, PRp, 16), jnp.float32),
      ),
  )(za, zb, disf, disr, W2, W2r)


def _apply2_body(count, gf, stf, w2, b2, ms2, gr, str_, w2r, b2r, ms2r, z2):
  a = jax.nn.relu(_gn_apply(gf[...], stf[...], w2[...], b2[...], ms2[...],
                            count))
  cc = jax.nn.relu(_gn_apply(gr[...], str_[...], w2r[...], b2r[...],
                             ms2r[...], count))
  z2[...] = a + cc


def _apply2_call(gf, stf, w2, b2, ms2, gr, str_, w2r, b2r, ms2r):
  nb = PRp // 1024
  vec = pl.BlockSpec((1, C2), lambda i: (0, 0))
  st = pl.BlockSpec((8, C2), lambda i: (0, 0))
  big = pl.BlockSpec((1024, C2), lambda i: (i, 0))
  return pl.pallas_call(
      functools.partial(_apply2_body, P),
      grid=(nb,),
      in_specs=[big, st, vec, vec, vec, big, st, vec, vec, vec],
      out_specs=big,
      out_shape=jax.ShapeDtypeStruct((PRp, C2), jnp.float32),
  )(gf, stf, w2, b2, ms2, gr, str_, w2r, b2r, ms2r)


def _final_body(zz, wp, bp_, o):
  y = zz[:, :32] * zz[:, 32:]
  o[...] = jnp.sum(y * wp[...], axis=1, keepdims=True) + bp_[...]


def _final_call(zz, Wp, bp_):
  nb = (K // 2) // 1024
  return pl.pallas_call(
      _final_body,
      grid=(nb,),
      in_specs=[
          pl.BlockSpec((1024, 64), lambda i: (i, 0)),
          pl.BlockSpec((1, C2), lambda i: (0, 0)),
          pl.BlockSpec((1, 1), lambda i: (0, 0)),
      ],
      out_specs=pl.BlockSpec((1024, 1), lambda i: (i, 0)),
      out_shape=jax.ShapeDtypeStruct((K // 2, 1), jnp.float32),
  )(zz, Wp, bp_)


def _pad_idx(a, total, fill):
  return jnp.concatenate(
      [a, jnp.full((total - a.shape[0],), fill, jnp.int32)]).reshape(-1, 128)


def _split_idx(a2d):
  return a2d.reshape(NW, -1, 128)


def kernel(x, edge1, pos, idx, ei2, emb, gn0_w, gn0_b, gn0_ms, W1, b1,
           gn1_w, gn1_b, gn1_ms, W2, b2, gn2_w, gn2_b, gn2_ms,
           W2r, b2r, gn2r_w, gn2r_b, gn2r_ms, Wp, bp):
  i32 = jnp.int32
  x = x.astype(i32)
  edge1 = edge1.astype(i32)
  pos = pos.astype(i32)
  idx = idx.astype(i32)
  ei2 = ei2.astype(i32)

  s1 = _pad_idx(edge1[0], EP, N)
  d1 = _pad_idx(edge1[1], EP, N)
  s2 = _pad_idx(ei2[0], EP, P)
  d2 = _pad_idx(ei2[1], EP, P)
  xp = _pad_idx(x, BN, 0)
  pa = _pad_idx(pos[:, 0], BP, 0)
  pb = _pad_idx(pos[:, 1], BP, 0)
  idxp = idx.reshape(-1, 128)

  zeros16 = jnp.zeros((PRp, 16), jnp.float32)
  zeros1d = jnp.zeros((PRp,), jnp.float32)

  g0w = gn0_w.reshape(1, C1)
  g0b = gn0_b.reshape(1, C1)
  g0ms = gn0_ms.reshape(1, C1)

  p1, p2, p3 = _deg_kernel(d1, d2, s2, zeros1d)
  dis1_2d, disf_2d, disr_2d = _dis_call(p1, p2, p3)
  dis1 = dis1_2d.reshape(NRp, 1)
  dis_f = disf_2d.reshape(PRp, 1)
  dis_r = disr_2d.reshape(PRp, 1)

  h0 = _make_gather(N, C1, BN, 13, 13)(emb, _split_idx(xp))[:NRp]
  st0 = _stats_call(h0, N, C1)
  t1 = _pack1_call(h0, st0, dis1, W1, g0w, g0b, g0ms)
  acc1 = _make_msg(NRp)(s1, d1, t1[0], t1[1], zeros16[:NRp])
  g1, st1 = _postagg_call(acc1, t1, dis1, b1.reshape(1, C2), NRp, N)
  h1 = _apply_call(g1, st1, gn1_w.reshape(1, C2), gn1_b.reshape(1, C2),
                   gn1_ms.reshape(1, C2), NRp, N)

  gather_pair = _make_gather(NRp, C2, BP, 25, 5)
  za = gather_pair(h1, _split_idx(pa))[:PRp]
  zb = gather_pair(h1, _split_idx(pb))[:PRp]
  tf, tr = _pack2_call(za, zb, dis_f, dis_r, W2, W2r)
  msg2 = _make_msg(PRp)
  accf = msg2(s2, d2, tf[0], tf[1], zeros16)
  accr = msg2(d2, s2, tr[0], tr[1], zeros16)
  g2f, stF = _postagg_call(accf, tf, dis_f, b2.reshape(1, C2), PRp, P)
  g2r, stR = _postagg_call(accr, tr, dis_r, b2r.reshape(1, C2), PRp, P)
  z2 = _apply2_call(g2f, stF, gn2_w.reshape(1, C2), gn2_b.reshape(1, C2),
                    gn2_ms.reshape(1, C2),
                    g2r, stR, gn2r_w.reshape(1, C2), gn2r_b.reshape(1, C2),
                    gn2r_ms.reshape(1, C2))

  zg = _make_gather(PRp, C2, K, 16, 8)(z2, _split_idx(idxp))
  out = _final_call(zg.reshape(K // 2, 64), Wp.T.reshape(1, C2),
                    bp.reshape(1, 1))
  return out


# speedup vs baseline: 27.2859x; 1.1065x over previous
import functools

import jax
import jax.numpy as jnp
from jax import lax
from jax.experimental import pallas as pl
from jax.experimental.pallas import tpu as pltpu
from jax.experimental.pallas import tpu_sc as plsc

N = 50000
E = 800000
P = 100000
K = 65536
C1 = 64
C2 = 32

NC = 2
NS = 16
NW = NC * NS

NRp = 53248
PRp = 102400
EP = 819200
EPS = 1e-5

_mesh = plsc.VectorSubcoreMesh(
    core_axis_name="c", subcore_axis_name="s", num_cores=NC, num_subcores=NS)
_sc_params = pltpu.CompilerParams(use_tc_tiling_on_sc=False)


def _make_gather(D, B, G, GG):
  bp = B // NW
  assert bp == G * 128 and G % GG == 0

  @functools.partial(
      pl.kernel,
      out_type=jax.ShapeDtypeStruct((B, D), jnp.float32),
      mesh=_mesh,
      compiler_params=_sc_params,
      scratch_types=[
          pltpu.VMEM((G, 128), jnp.int32),
          pltpu.VMEM((bp, D), jnp.float32),
          pltpu.SemaphoreType.DMA,
      ],
  )
  def k(table, idx3d, out, idxv, rows, sem):
    c = lax.axis_index("c")
    s = lax.axis_index("s")
    wid = c * NS + s
    pltpu.sync_copy(idx3d.at[wid], idxv)

    def grp(ii, _):
      for j in range(GG):
        jj = ii * GG + j
        pltpu.async_copy(table.at[idxv.at[jj]],
                         rows.at[pl.ds(jj * 128, 128)], sem)
      for j in range(GG):
        jj = ii * GG + j
        pltpu.make_async_copy(table.at[idxv.at[jj]],
                              rows.at[pl.ds(jj * 128, 128)], sem).wait()
      return 0

    lax.fori_loop(0, G // GG, grp, 0)
    pltpu.sync_copy(rows, out.at[pl.ds(wid * bp, bp)])

  return k


@functools.partial(
    pl.kernel,
    out_type=(
        jax.ShapeDtypeStruct((NC, NRp), jnp.float32),
        jax.ShapeDtypeStruct((NC, PRp), jnp.float32),
        jax.ShapeDtypeStruct((NC, PRp), jnp.float32),
    ),
    mesh=_mesh,
    compiler_params=_sc_params,
    scratch_types=[
        pltpu.VMEM_SHARED((NRp,), jnp.float32),
        pltpu.VMEM_SHARED((PRp,), jnp.float32),
        pltpu.VMEM_SHARED((PRp,), jnp.float32),
        pltpu.VMEM((8, 128), jnp.int32),
        pltpu.VMEM((128,), jnp.float32),
        pltpu.SemaphoreType.DMA,
    ],
)
def _deg_kernel(d1, d2, d3, zeros1d, p1, p2, p3, a1, a2, a3,
                didx, ones, sem):
  c = lax.axis_index("c")
  s = lax.axis_index("s")
  wid = c * NS + s
  n1 = NRp // NS
  n2 = PRp // NS
  for i in range(8):
    ones[pl.ds(i * 16, 16)] = jnp.ones((16,), jnp.float32)
  pltpu.sync_copy(zeros1d.at[pl.ds(s * n1, n1)], a1.at[pl.ds(s * n1, n1)])
  pltpu.sync_copy(zeros1d.at[pl.ds(s * n2, n2)], a2.at[pl.ds(s * n2, n2)])
  pltpu.sync_copy(zeros1d.at[pl.ds(s * n2, n2)], a3.at[pl.ds(s * n2, n2)])
  plsc.subcore_barrier()

  nrow_pt = EP // 128 // NW

  for (dref, aref) in ((d1, a1), (d2, a2), (d3, a3)):
    def chunk(g, _, dref=dref, aref=aref):
      rb = wid * nrow_pt + g * 8
      pltpu.sync_copy(dref.at[pl.ds(rb, 8)], didx)
      for j in range(8):
        pltpu.async_copy(ones, aref.at[didx.at[j]], sem, add=True)
      for j in range(8):
        pltpu.make_async_copy(ones, aref.at[didx.at[j]], sem).wait()
      return 0

    lax.fori_loop(0, nrow_pt // 8, chunk, 0)

  plsc.subcore_barrier()
  pltpu.sync_copy(a1.at[pl.ds(s * n1, n1)], p1.at[c, pl.ds(s * n1, n1)])
  pltpu.sync_copy(a2.at[pl.ds(s * n2, n2)], p2.at[c, pl.ds(s * n2, n2)])
  pltpu.sync_copy(a3.at[pl.ds(s * n2, n2)], p3.at[c, pl.ds(s * n2, n2)])


def _make_msg(nrows):
  rpt = nrows // NS
  nrow_pt = EP // 128 // NS

  @functools.partial(
      pl.kernel,
      out_type=jax.ShapeDtypeStruct((NC, nrows, 16), jnp.float32),
      mesh=_mesh,
      compiler_params=_sc_params,
      scratch_types=[
          pltpu.VMEM_SHARED((nrows, 16), jnp.float32),
          pltpu.VMEM((8, 128), jnp.int32),
          pltpu.VMEM((8, 128), jnp.int32),
          pltpu.VMEM((1024, 16), jnp.float32),
          pltpu.SemaphoreType.DMA,
          pltpu.SemaphoreType.DMA,
      ],
  )
  def k(src2d, dst2d, table0, table1, zeros, out, acc,
        sidx, didx, msg, semg, sems):
    c = lax.axis_index("c")
    s = lax.axis_index("s")
    pltpu.sync_copy(zeros.at[pl.ds(s * rpt, rpt)], acc.at[pl.ds(s * rpt, rpt)])
    plsc.subcore_barrier()

    def edge_loop(table):
      def chunk(g, _):
        rb = s * nrow_pt + g * 8
        pltpu.sync_copy(src2d.at[pl.ds(rb, 8)], sidx)
        pltpu.sync_copy(dst2d.at[pl.ds(rb, 8)], didx)
        for j in range(8):
          pltpu.async_copy(table.at[sidx.at[j]],
                           msg.at[pl.ds(j * 128, 128)], semg)
        for j in range(8):
          pltpu.make_async_copy(table.at[sidx.at[j]],
                                msg.at[pl.ds(j * 128, 128)], semg).wait()
        for j in range(8):
          pltpu.async_copy(msg.at[pl.ds(j * 128, 128)],
                           acc.at[didx.at[j]], sems, add=True)
        for j in range(8):
          pltpu.make_async_copy(msg.at[pl.ds(j * 128, 128)],
                                acc.at[didx.at[j]], sems).wait()
        return 0

      lax.fori_loop(0, nrow_pt // 8, chunk, 0)

    @pl.when(c == 0)
    def _():
      edge_loop(table0)

    @pl.when(c == 1)
    def _():
      edge_loop(table1)

    plsc.subcore_barrier()
    pltpu.sync_copy(acc.at[pl.ds(s * rpt, rpt)],
                    out.at[c, pl.ds(s * rpt, rpt)])

  return k


def _dis_body(d1, d2, d3, o1, o2, o3):
  o1[...] = lax.rsqrt(d1[0:1] + d1[1:2] + 1.0)
  o2[...] = lax.rsqrt(d2[0:1] + d2[1:2] + 1.0)
  o3[...] = lax.rsqrt(d3[0:1] + d3[1:2] + 1.0)


def _dis_call(p1, p2, p3):
  return pl.pallas_call(
      _dis_body,
      out_shape=(
          jax.ShapeDtypeStruct((1, NRp), jnp.float32),
          jax.ShapeDtypeStruct((1, PRp), jnp.float32),
          jax.ShapeDtypeStruct((1, PRp), jnp.float32),
      ),
  )(p1, p2, p3)


def _stats_body(count, xr, st):
  i = pl.program_id(0)
  blk = xr[...]
  rows = i * 1024 + lax.broadcasted_iota(jnp.int32, (1024, 1), 0)
  m = (rows < count).astype(jnp.float32)
  xm = blk * m
  s1 = jnp.sum(xm, axis=0, keepdims=True)
  s2 = jnp.sum(xm * xm, axis=0, keepdims=True)

  @pl.when(i == 0)
  def _():
    st[...] = jnp.zeros_like(st)

  st[0:1, :] += s1
  st[1:2, :] += s2


def _stats_call(xr, count, C):
  nb = xr.shape[0] // 1024
  return pl.pallas_call(
      functools.partial(_stats_body, count),
      grid=(nb,),
      in_specs=[pl.BlockSpec((1024, C), lambda i: (i, 0))],
      out_specs=pl.BlockSpec((8, C), lambda i: (0, 0)),
      out_shape=jax.ShapeDtypeStruct((8, C), jnp.float32),
  )(xr)


def _gn_apply(x, st, w, b, ms, count):
  m = st[0:1, :] / count
  msq = st[1:2, :] / count
  var = msq - 2.0 * ms * m * m + (ms * m) * (ms * m)
  return w * (x - ms * m) * lax.rsqrt(var + EPS) + b


def _pack1_body(h0, st0, dis, W1, g0w, g0b, g0ms, t1a, t1b):
  hn = _gn_apply(h0[...], st0[...], g0w[...], g0b[...], g0ms[...], N)
  hW = jnp.dot(hn, W1[...], preferred_element_type=jnp.float32)
  hs = jnp.transpose(dis[...]) * hW
  t1a[...] = hs[:, :16]
  t1b[...] = hs[:, 16:]


def _pack1_call(h0, st0, dis1, W1, g0w, g0b, g0ms):
  nb = NRp // 1024
  full = lambda r, c: pl.BlockSpec((r, c), lambda i: (0, 0))
  return pl.pallas_call(
      _pack1_body,
      grid=(nb,),
      in_specs=[
          pl.BlockSpec((1024, C1), lambda i: (i, 0)),
          full(8, C1),
          pl.BlockSpec((1, 1024), lambda i: (0, i)),
          full(C1, C2),
          full(1, C1),
          full(1, C1),
          full(1, C1),
      ],
      out_specs=(
          pl.BlockSpec((1024, 16), lambda i: (i, 0)),
          pl.BlockSpec((1024, 16), lambda i: (i, 0)),
      ),
      out_shape=(
          jax.ShapeDtypeStruct((NRp, 16), jnp.float32),
          jax.ShapeDtypeStruct((NRp, 16), jnp.float32),
      ),
  )(h0, st0, dis1, W1, g0w, g0b, g0ms)


def _postagg1_body(acc, ta, tb, dis, bias, g, st):
  i = pl.program_id(0)
  acc32 = jnp.concatenate([acc[0], acc[1]], axis=1)
  hs32 = jnp.concatenate([ta[...], tb[...]], axis=1)
  gv = jnp.transpose(dis[...]) * (acc32 + hs32) + bias[...]
  g[...] = gv
  rows = i * 1024 + lax.broadcasted_iota(jnp.int32, (1024, 1), 0)
  gm = gv * (rows < N).astype(jnp.float32)

  @pl.when(i == 0)
  def _():
    st[...] = jnp.zeros_like(st)

  st[0:1, :] += jnp.sum(gm, axis=0, keepdims=True)
  st[1:2, :] += jnp.sum(gm * gm, axis=0, keepdims=True)


def _postagg1_call(acc, ta, tb, dis, bias):
  nb = NRp // 1024
  return pl.pallas_call(
      _postagg1_body,
      grid=(nb,),
      in_specs=[
          pl.BlockSpec((2, 1024, 16), lambda i: (0, i, 0)),
          pl.BlockSpec((1024, 16), lambda i: (i, 0)),
          pl.BlockSpec((1024, 16), lambda i: (i, 0)),
          pl.BlockSpec((1, 1024), lambda i: (0, i)),
          pl.BlockSpec((1, C2), lambda i: (0, 0)),
      ],
      out_specs=(
          pl.BlockSpec((1024, C2), lambda i: (i, 0)),
          pl.BlockSpec((8, C2), lambda i: (0, 0)),
      ),
      out_shape=(
          jax.ShapeDtypeStruct((NRp, C2), jnp.float32),
          jax.ShapeDtypeStruct((8, C2), jnp.float32),
      ),
  )(acc, ta, tb, dis, bias)


def _apply_body(count, g, st, w, b, ms, h):
  h[...] = jax.nn.relu(_gn_apply(g[...], st[...], w[...], b[...], ms[...],
                                 count))


def _apply_call(g, st, w, b, ms, nrows, count):
  nb = nrows // 1024
  vec = pl.BlockSpec((1, C2), lambda i: (0, 0))
  return pl.pallas_call(
      functools.partial(_apply_body, count),
      grid=(nb,),
      in_specs=[
          pl.BlockSpec((1024, C2), lambda i: (i, 0)),
          pl.BlockSpec((8, C2), lambda i: (0, 0)),
          vec, vec, vec,
      ],
      out_specs=pl.BlockSpec((1024, C2), lambda i: (i, 0)),
      out_shape=jax.ShapeDtypeStruct((nrows, C2), jnp.float32),
  )(g, st, w, b, ms)


def _pack2_body(za, zb, disf, disr, W2, W2r, tfa, tfb, tra, trb):
  z = za[...] * zb[...]
  zf = jnp.transpose(disf[...]) * jnp.dot(z, W2[...],
                                          preferred_element_type=jnp.float32)
  zr = jnp.transpose(disr[...]) * jnp.dot(z, W2r[...],
                                          preferred_element_type=jnp.float32)
  tfa[...] = zf[:, :16]
  tfb[...] = zf[:, 16:]
  tra[...] = zr[:, :16]
  trb[...] = zr[:, 16:]


def _pack2_call(za, zb, disf, disr, W2, W2r):
  nb = PRp // 1024
  half = lambda: pl.BlockSpec((1024, 16), lambda i: (i, 0))
  t16 = jax.ShapeDtypeStruct((PRp, 16), jnp.float32)
  return pl.pallas_call(
      _pack2_body,
      grid=(nb,),
      in_specs=[
          pl.BlockSpec((1024, C2), lambda i: (i, 0)),
          pl.BlockSpec((1024, C2), lambda i: (i, 0)),
          pl.BlockSpec((1, 1024), lambda i: (0, i)),
          pl.BlockSpec((1, 1024), lambda i: (0, i)),
          pl.BlockSpec((C2, C2), lambda i: (0, 0)),
          pl.BlockSpec((C2, C2), lambda i: (0, 0)),
      ],
      out_specs=(half(), half(), half(), half()),
      out_shape=(t16, t16, t16, t16),
  )(za, zb, disf, disr, W2, W2r)


def _postagg2_body(accf, tfa, tfb, disf, b2,
                   accr, tra, trb, disr, b2r, gf, gr, stf, str_):
  i = pl.program_id(0)
  rows = i * 1024 + lax.broadcasted_iota(jnp.int32, (1024, 1), 0)
  m = (rows < P).astype(jnp.float32)

  @pl.when(i == 0)
  def _():
    stf[...] = jnp.zeros_like(stf)
    str_[...] = jnp.zeros_like(str_)

  a32 = jnp.concatenate([accf[0], accf[1]], axis=1)
  h32 = jnp.concatenate([tfa[...], tfb[...]], axis=1)
  gv = jnp.transpose(disf[...]) * (a32 + h32) + b2[...]
  gf[...] = gv
  gm = gv * m
  stf[0:1, :] += jnp.sum(gm, axis=0, keepdims=True)
  stf[1:2, :] += jnp.sum(gm * gm, axis=0, keepdims=True)

  a32 = jnp.concatenate([accr[0], accr[1]], axis=1)
  h32 = jnp.concatenate([tra[...], trb[...]], axis=1)
  gv = jnp.transpose(disr[...]) * (a32 + h32) + b2r[...]
  gr[...] = gv
  gm = gv * m
  str_[0:1, :] += jnp.sum(gm, axis=0, keepdims=True)
  str_[1:2, :] += jnp.sum(gm * gm, axis=0, keepdims=True)


def _postagg2_call(accf, tfa, tfb, disf, b2, accr, tra, trb, disr, b2r):
  nb = PRp // 1024
  acc_s = pl.BlockSpec((2, 1024, 16), lambda i: (0, i, 0))
  t_s = pl.BlockSpec((1024, 16), lambda i: (i, 0))
  d_s = pl.BlockSpec((1, 1024), lambda i: (0, i))
  b_s = pl.BlockSpec((1, C2), lambda i: (0, 0))
  g_s = pl.BlockSpec((1024, C2), lambda i: (i, 0))
  st_s = pl.BlockSpec((8, C2), lambda i: (0, 0))
  return pl.pallas_call(
      _postagg2_body,
      grid=(nb,),
      in_specs=[acc_s, t_s, t_s, d_s, b_s, acc_s, t_s, t_s, d_s, b_s],
      out_specs=(g_s, g_s, st_s, st_s),
      out_shape=(
          jax.ShapeDtypeStruct((PRp, C2), jnp.float32),
          jax.ShapeDtypeStruct((PRp, C2), jnp.float32),
          jax.ShapeDtypeStruct((8, C2), jnp.float32),
          jax.ShapeDtypeStruct((8, C2), jnp.float32),
      ),
  )(accf, tfa, tfb, disf, b2, accr, tra, trb, disr, b2r)


def _apply2_body(count, gf, stf, w2, b2, ms2, gr, str_, w2r, b2r, ms2r, z2):
  a = jax.nn.relu(_gn_apply(gf[...], stf[...], w2[...], b2[...], ms2[...],
                            count))
  cc = jax.nn.relu(_gn_apply(gr[...], str_[...], w2r[...], b2r[...],
                             ms2r[...], count))
  z2[...] = a + cc


def _apply2_call(gf, stf, w2, b2, ms2, gr, str_, w2r, b2r, ms2r):
  nb = PRp // 1024
  vec = pl.BlockSpec((1, C2), lambda i: (0, 0))
  st = pl.BlockSpec((8, C2), lambda i: (0, 0))
  big = pl.BlockSpec((1024, C2), lambda i: (i, 0))
  return pl.pallas_call(
      functools.partial(_apply2_body, P),
      grid=(nb,),
      in_specs=[big, st, vec, vec, vec, big, st, vec, vec, vec],
      out_specs=big,
      out_shape=jax.ShapeDtypeStruct((PRp, C2), jnp.float32),
  )(gf, stf, w2, b2, ms2, gr, str_, w2r, b2r, ms2r)


def _final_body(zga, zgb, wp, bp_, o):
  y = zga[...] * zgb[...]
  o[...] = jnp.sum(y * wp[...], axis=1, keepdims=True) + bp_[...]


def _final_call(zga, zgb, Wp, bp_):
  nb = (K // 2) // 1024
  return pl.pallas_call(
      _final_body,
      grid=(nb,),
      in_specs=[
          pl.BlockSpec((1024, C2), lambda i: (i, 0)),
          pl.BlockSpec((1024, C2), lambda i: (i, 0)),
          pl.BlockSpec((1, C2), lambda i: (0, 0)),
          pl.BlockSpec((1, 1), lambda i: (0, 0)),
      ],
      out_specs=pl.BlockSpec((1024, 1), lambda i: (i, 0)),
      out_shape=jax.ShapeDtypeStruct((K // 2, 1), jnp.float32),
  )(zga, zgb, Wp, bp_)


def _pad_idx(a, total, fill):
  if total > a.shape[0]:
    a = jnp.concatenate([a, jnp.full((total - a.shape[0],), fill, jnp.int32)])
  return a.reshape(NW, -1, 128)


def kernel(x, edge1, pos, idx, ei2, emb, gn0_w, gn0_b, gn0_ms, W1, b1,
           gn1_w, gn1_b, gn1_ms, W2, b2, gn2_w, gn2_b, gn2_ms,
           W2r, b2r, gn2r_w, gn2r_b, gn2r_ms, Wp, bp):
  i32 = jnp.int32
  x = x.astype(i32)
  edge1 = edge1.astype(i32)
  pos = pos.astype(i32)
  idx = idx.astype(i32)
  ei2 = ei2.astype(i32)

  def pad_edge(a, fill):
    return jnp.concatenate(
        [a, jnp.full((EP - a.shape[0],), fill, i32)]).reshape(-1, 128)

  s1 = pad_edge(edge1[0], N)
  d1 = pad_edge(edge1[1], N)
  s2 = pad_edge(ei2[0], P)
  d2 = pad_edge(ei2[1], P)
  xp = _pad_idx(x, NRp, 0)
  pa = _pad_idx(pos[:, 0], PRp, 0)
  pb = _pad_idx(pos[:, 1], PRp, 0)
  ia = _pad_idx(idx[0::2], K // 2, 0)
  ib = _pad_idx(idx[1::2], K // 2, 0)

  zeros16 = jnp.zeros((PRp, 16), jnp.float32)
  zeros1d = jnp.zeros((PRp,), jnp.float32)

  p1, p2, p3 = _deg_kernel(d1, d2, s2, zeros1d)
  dis1, dis_f, dis_r = _dis_call(p1, p2, p3)

  h0 = _make_gather(C1, NRp, 13, 13)(emb, xp)
  st0 = _stats_call(h0, N, C1)
  t1a, t1b = _pack1_call(h0, st0, dis1, W1, gn0_w.reshape(1, C1),
                         gn0_b.reshape(1, C1), gn0_ms.reshape(1, C1))
  acc1 = _make_msg(NRp)(s1, d1, t1a, t1b, zeros16[:NRp])
  g1, st1 = _postagg1_call(acc1, t1a, t1b, dis1, b1.reshape(1, C2))
  h1 = _apply_call(g1, st1, gn1_w.reshape(1, C2), gn1_b.reshape(1, C2),
                   gn1_ms.reshape(1, C2), NRp, N)

  gather_pair = _make_gather(C2, PRp, 25, 5)
  za = gather_pair(h1, pa)
  zb = gather_pair(h1, pb)
  tfa, tfb, tra, trb = _pack2_call(za, zb, dis_f, dis_r, W2, W2r)
  msg2 = _make_msg(PRp)
  accf = msg2(s2, d2, tfa, tfb, zeros16)
  accr = msg2(d2, s2, tra, trb, zeros16)
  g2f, g2r, stF, stR = _postagg2_call(
      accf, tfa, tfb, dis_f, b2.reshape(1, C2),
      accr, tra, trb, dis_r, b2r.reshape(1, C2))
  z2 = _apply2_call(g2f, stF, gn2_w.reshape(1, C2), gn2_b.reshape(1, C2),
                    gn2_ms.reshape(1, C2),
                    g2r, stR, gn2r_w.reshape(1, C2), gn2r_b.reshape(1, C2),
                    gn2r_ms.reshape(1, C2))

  gather_fin = _make_gather(C2, K // 2, 8, 8)
  zga = gather_fin(z2, ia)
  zgb = gather_fin(z2, ib)
  out = _final_call(zga, zgb, Wp.T.reshape(1, C2), bp.reshape(1, 1))
  return out
